# baseline (speedup 1.0000x reference)
import functools

import jax
import jax.numpy as jnp
from jax import lax
from jax.experimental import pallas as pl
from jax.experimental.pallas import tpu as pltpu
from jax.experimental.pallas import tpu_sc as plsc

_BATCH = 16384
_FIELDS = 100
_LANES = 16
_NUM_WORKERS = 32
_ROWS_W = _BATCH // _NUM_WORKERS
_FULL = _FIELDS // _LANES
_TAIL_OFF = _FIELDS - _LANES
_NCHUNK = 4
_CROWS = _ROWS_W // _NCHUNK
_CIDX = _CROWS * _FIELDS
_UNROLL = 8
_RSTEPS = _CIDX // (_LANES * _UNROLL)

_mesh = plsc.VectorSubcoreMesh(core_axis_name="c", subcore_axis_name="s")


@functools.partial(
    pl.kernel,
    mesh=_mesh,
    out_type=jax.ShapeDtypeStruct((_NUM_WORKERS, _LANES), jnp.float32),
    scratch_types=[
        pltpu.VMEM((_CROWS, _FIELDS), jnp.int32),
        pltpu.VMEM((_CROWS, _FIELDS), jnp.int32),
        pltpu.VMEM((_CIDX,), jnp.int32),
        pltpu.VMEM((_CIDX,), jnp.int32),
        pltpu.VMEM((_CIDX,), jnp.float32),
        pltpu.VMEM((_CIDX,), jnp.float32),
        pltpu.VMEM((_LANES,), jnp.float32),
        pltpu.SemaphoreType.DMA,
        pltpu.SemaphoreType.DMA,
        pltpu.SemaphoreType.DMA,
        pltpu.SemaphoreType.DMA,
    ],
)
def _gather_sum(idx_hbm, w_hbm, out_hbm, stage0, stage1, flat0, flat1,
                vals0, vals1, acc_v, ssem0, ssem1, gsem0, gsem1):
    nc = plsc.get_sparse_core_info().num_cores
    wid = lax.axis_index("s") * nc + lax.axis_index("c")
    row0 = wid * _ROWS_W
    stages = (stage0, stage1)
    flats = (flat0, flat1)
    vals = (vals0, vals1)
    ssems = (ssem0, ssem1)
    gsems = (gsem0, gsem1)

    def start_stage(c):
        b = c % 2
        return pltpu.async_copy(
            idx_hbm.at[pl.ds(row0 + c * _CROWS, _CROWS), :], stages[b],
            ssems[b])

    def repack(c):
        b = c % 2
        stage, flat = stages[b], flats[b]

        def body(r, carry):
            o = r * _FIELDS
            for k in range(_FULL):
                flat[pl.ds(o + k * _LANES, _LANES)] = (
                    stage[r, pl.ds(k * _LANES, _LANES)])
            flat[pl.ds(o + _TAIL_OFF, _LANES)] = (
                stage[r, pl.ds(_TAIL_OFF, _LANES)])
            return carry

        lax.fori_loop(0, _CROWS, body, 0)

    def start_gather(c):
        b = c % 2
        return pltpu.async_copy(w_hbm.at[flats[b]], vals[b], gsems[b])

    def reduce_chunk(c, accs):
        v = vals[c % 2]

        def body(i, a):
            o = i * (_LANES * _UNROLL)
            return tuple(
                a[j] + v[pl.ds(o + j * _LANES, _LANES)]
                for j in range(_UNROLL)
            )

        return lax.fori_loop(0, _RSTEPS, body, accs)

    h_s = [None] * _NCHUNK
    h_g = [None] * _NCHUNK
    h_s[0] = start_stage(0)
    h_s[1] = start_stage(1)
    h_s[0].wait()
    repack(0)
    h_g[0] = start_gather(0)
    h_s[2] = start_stage(2)
    h_s[1].wait()
    repack(1)
    h_g[1] = start_gather(1)
    h_s[3] = start_stage(3)

    accs = (jnp.zeros((_LANES,), jnp.float32),) * _UNROLL
    for c in range(2, _NCHUNK):
        h_g[c - 2].wait()
        accs = reduce_chunk(c - 2, accs)
        h_s[c].wait()
        repack(c)
        h_g[c] = start_gather(c)
    h_g[_NCHUNK - 2].wait()
    accs = reduce_chunk(_NCHUNK - 2, accs)
    h_g[_NCHUNK - 1].wait()
    accs = reduce_chunk(_NCHUNK - 1, accs)

    total = accs[0]
    for j in range(1, _UNROLL):
        total = total + accs[j]
    acc_v[...] = total
    pltpu.sync_copy(acc_v, out_hbm.at[wid])


def kernel(non_zero_indices, weights):
    partials = _gather_sum(non_zero_indices, weights)
    return jnp.sum(partials)


# speedup vs baseline: 1.0368x; 1.0368x over previous
import functools

import jax
import jax.numpy as jnp
from jax import lax
from jax.experimental import pallas as pl
from jax.experimental.pallas import tpu as pltpu
from jax.experimental.pallas import tpu_sc as plsc

_BATCH = 16384
_FIELDS = 100
_LANES = 16
_NUM_WORKERS = 32
_ROWS_W = _BATCH // _NUM_WORKERS
_FULL = _FIELDS // _LANES
_TAIL_OFF = _FIELDS - _LANES
_TAIL_DUP = _LANES - (_FIELDS - _FULL * _LANES)

_mesh = plsc.VectorSubcoreMesh(core_axis_name="c", subcore_axis_name="s")


@functools.partial(
    pl.kernel,
    mesh=_mesh,
    out_type=jax.ShapeDtypeStruct((_NUM_WORKERS, _LANES), jnp.float32),
    compiler_params=pltpu.CompilerParams(use_tc_tiling_on_sc=True),
    scratch_types=[
        pltpu.VMEM((_ROWS_W // 2, _FIELDS), jnp.int32),
        pltpu.VMEM((_ROWS_W // 2, _FIELDS), jnp.float32),
        pltpu.VMEM((_LANES,), jnp.float32),
        pltpu.SemaphoreType.DMA,
    ],
)
def _gather_sum(idx_hbm, w_hbm, out_hbm, idx_v, vals_v, acc_v, sem):
    nc = plsc.get_sparse_core_info().num_cores
    wid = lax.axis_index("s") * nc + lax.axis_index("c")
    half = _ROWS_W // 2

    tail_mask = lax.iota(jnp.int32, _LANES) < _TAIL_DUP
    fzero = jnp.zeros((_LANES,), jnp.float32)
    accs = (fzero,) * (_FULL + 1)

    for h in range(2):
        pltpu.sync_copy(
            idx_hbm.at[pl.ds(wid * _ROWS_W + h * half, half), :], idx_v)

        def issue(r, carry):
            pltpu.async_copy(w_hbm.at[idx_v.at[r]], vals_v.at[r], sem)
            return carry

        lax.fori_loop(0, half, issue, 0)

        def drain(r, carry):
            pltpu.make_async_copy(
                w_hbm.at[idx_v.at[r]], vals_v.at[r], sem).wait()
            return carry

        lax.fori_loop(0, half, drain, 0)

        def body(r, a):
            new = [a[j] + vals_v[r, pl.ds(j * _LANES, _LANES)]
                   for j in range(_FULL)]
            tail = vals_v[r, pl.ds(_TAIL_OFF, _LANES)]
            new.append(a[_FULL] + jnp.where(tail_mask, fzero, tail))
            return tuple(new)

        accs = lax.fori_loop(0, half, body, accs)

    total = accs[0]
    for j in range(1, _FULL + 1):
        total = total + accs[j]
    acc_v[...] = total
    pltpu.sync_copy(acc_v, out_hbm.at[wid])


def kernel(non_zero_indices, weights):
    partials = _gather_sum(non_zero_indices, weights)
    return jnp.sum(partials)


# speedup vs baseline: 1.1114x; 1.0720x over previous
import functools

import jax
import jax.numpy as jnp
from jax import lax
from jax.experimental import pallas as pl
from jax.experimental.pallas import tpu as pltpu
from jax.experimental.pallas import tpu_sc as plsc

_BATCH = 16384
_FIELDS = 100
_LANES = 16
_NUM_WORKERS = 32
_COLS_W = _BATCH // _NUM_WORKERS
_SEG = 128
_NSEG = _COLS_W // _SEG
_NSTREAMS = _FIELDS * _NSEG
_UNROLL = _SEG // _LANES

_mesh = plsc.VectorSubcoreMesh(core_axis_name="c", subcore_axis_name="s")


@functools.partial(
    pl.kernel,
    mesh=_mesh,
    out_type=jax.ShapeDtypeStruct((_NUM_WORKERS, _LANES), jnp.float32),
    compiler_params=pltpu.CompilerParams(use_tc_tiling_on_sc=True),
    scratch_types=[
        pltpu.VMEM((_FIELDS, _NSEG, _SEG), jnp.int32),
        pltpu.VMEM((_FIELDS, _NSEG, _SEG), jnp.float32),
        pltpu.VMEM((_LANES,), jnp.float32),
        pltpu.SemaphoreType.DMA,
    ],
)
def _gather_sum(idx_hbm, w_hbm, out_hbm, idx_v, vals_v, acc_v, sem):
    nc = plsc.get_sparse_core_info().num_cores
    wid = lax.axis_index("s") * nc + lax.axis_index("c")
    col0 = wid * _COLS_W

    for k in range(_NSEG):
        pltpu.sync_copy(
            idx_hbm.at[:, pl.ds(col0 + k * _SEG, _SEG)], idx_v.at[:, k, :])

    def issue(q, carry):
        r = q // _NSEG
        k = lax.rem(q, _NSEG)
        pltpu.async_copy(w_hbm.at[idx_v.at[r, k]], vals_v.at[r, k], sem)
        return carry

    lax.fori_loop(0, _NSTREAMS, issue, 0)

    def drain(q, carry):
        r = q // _NSEG
        k = lax.rem(q, _NSEG)
        pltpu.make_async_copy(
            w_hbm.at[idx_v.at[r, k]], vals_v.at[r, k], sem).wait()
        return carry

    lax.fori_loop(0, _NSTREAMS, drain, 0)

    def body(q, accs):
        r = q // _NSEG
        k = lax.rem(q, _NSEG)
        return tuple(
            accs[j] + vals_v[r, k, pl.ds(j * _LANES, _LANES)]
            for j in range(_UNROLL)
        )

    zeros = jnp.zeros((_LANES,), jnp.float32)
    accs = lax.fori_loop(0, _NSTREAMS, body, (zeros,) * _UNROLL)
    total = accs[0]
    for j in range(1, _UNROLL):
        total = total + accs[j]
    acc_v[...] = total
    pltpu.sync_copy(acc_v, out_hbm.at[wid])


def kernel(non_zero_indices, weights):
    partials = _gather_sum(non_zero_indices.T, weights)
    return jnp.sum(partials)
